# baseline (speedup 1.0000x reference)
import jax
import jax.numpy as jnp
from jax import lax
from jax.experimental import pallas as pl
from jax.experimental.pallas import tpu as pltpu


def _dropblock_body(mask_ref, x_ref, o_ref, m_ref, *, bs, H, W):
    HW = H * W

    @pl.when(pl.program_id(0) == 0)
    def _():
        m = mask_ref[:]
        xcol = lax.broadcasted_iota(jnp.int32, (1, HW), 1) & (W - 1)
        r = m
        for j in range(1, bs):
            sh = jnp.pad(m, ((0, 0), (j, 0)))[:, :HW]
            r = jnp.maximum(r, jnp.where(xcol >= j, sh, 0.0))
        b = r
        for i in range(1, bs):
            sh = jnp.pad(r, ((0, 0), (W * i, 0)))[:, :HW]
            b = jnp.maximum(b, sh)
        bm = 1.0 - b
        scale = jnp.float32(HW) / jnp.sum(bm)
        m_ref[:] = jnp.broadcast_to((bm * scale)[:, None, :], (1, 8, HW))

    o_ref[:] = x_ref[:] * m_ref[:]


def kernel(x, block_size, feat_size, drop_rate):
    B, C, H, W = x.shape
    bs = 7
    gamma = drop_rate / (block_size ** 2) * (
        (feat_size ** 2) / ((feat_size - block_size + 1) ** 2)
    )
    mkey = jax.random.fold_in(jax.random.key(0), 1)
    mask = jax.random.bernoulli(mkey, gamma, (H, W)).astype(jnp.float32)

    HW = H * W
    R = (B * C) // 8
    xr = x.reshape(R, 8, HW)
    K = 32
    G = R // K

    out = pl.pallas_call(
        lambda mask_ref, x_ref, o_ref, m_ref: _dropblock_body(
            mask_ref, x_ref, o_ref, m_ref, bs=bs, H=H, W=W
        ),
        grid=(G,),
        in_specs=[
            pl.BlockSpec((1, HW), lambda i: (0, 0)),
            pl.BlockSpec((K, 8, HW), lambda i: (i, 0, 0)),
        ],
        out_specs=pl.BlockSpec((K, 8, HW), lambda i: (i, 0, 0)),
        out_shape=jax.ShapeDtypeStruct((R, 8, HW), x.dtype),
        scratch_shapes=[pltpu.VMEM((1, 8, HW), jnp.float32)],
        compiler_params=pltpu.CompilerParams(
            dimension_semantics=("arbitrary",),
        ),
    )(mask.reshape(1, HW), xr)
    return out.reshape(B, C, H, W)


# speedup vs baseline: 1.8198x; 1.8198x over previous
import jax
import jax.numpy as jnp
from jax.experimental import pallas as pl
from jax.experimental.pallas import tpu as pltpu


def _dropblock_body(mask_ref, x_ref, o_ref, m_ref, *, bs, H, W):
    @pl.when(pl.program_id(0) == 0)
    def _():
        m = mask_ref[:]
        pw = jnp.pad(m, ((0, 0), (bs - 1, 0)))
        r = m
        for j in range(1, bs):
            r = jnp.maximum(r, pw[:, bs - 1 - j : bs - 1 - j + W])
        ph = jnp.pad(r, ((bs - 1, 0), (0, 0)))
        b = r
        for i in range(1, bs):
            b = jnp.maximum(b, ph[bs - 1 - i : bs - 1 - i + H, :])
        bm = 1.0 - b
        scale = jnp.float32(H * W) / jnp.sum(bm)
        m_ref[:] = bm * scale

    o_ref[:] = x_ref[:] * m_ref[:][None, :, :]


def kernel(x, block_size, feat_size, drop_rate):
    B, C, H, W = x.shape
    bs = 7
    gamma = drop_rate / (block_size ** 2) * (
        (feat_size ** 2) / ((feat_size - block_size + 1) ** 2)
    )
    mkey = jax.random.fold_in(jax.random.key(0), 1)
    mask = jax.random.bernoulli(mkey, gamma, (H, W)).astype(jnp.float32)

    xr = x.reshape(B * C, H, W)
    K = 256
    G = (B * C) // K

    out = pl.pallas_call(
        lambda mask_ref, x_ref, o_ref, m_ref: _dropblock_body(
            mask_ref, x_ref, o_ref, m_ref, bs=bs, H=H, W=W
        ),
        grid=(G,),
        in_specs=[
            pl.BlockSpec((H, W), lambda i: (0, 0)),
            pl.BlockSpec((K, H, W), lambda i: (i, 0, 0)),
        ],
        out_specs=pl.BlockSpec((K, H, W), lambda i: (i, 0, 0)),
        out_shape=jax.ShapeDtypeStruct((B * C, H, W), x.dtype),
        scratch_shapes=[pltpu.VMEM((H, W), jnp.float32)],
        compiler_params=pltpu.CompilerParams(
            dimension_semantics=("arbitrary",),
        ),
    )(mask, xr)
    return out.reshape(B, C, H, W)
